# Initial kernel scaffold; baseline (speedup 1.0000x reference)
import jax
import jax.numpy as jnp
from jax.experimental import pallas as pl


def kernel(x, W, b):
    raise NotImplementedError("write your pallas kernel here")


# speedup vs baseline: 1.0028x; 1.0028x over previous
import jax
import jax.numpy as jnp
from jax.experimental import pallas as pl
from jax.experimental.pallas import tpu as pltpu

B = 4096
I = 4096
NCORES = 2
BLK = 512
KC = (I // NCORES) // BLK


def _body(x_ref, w_ref, b_ref, o_ref):
    k = pl.program_id(1)
    wsum = jnp.sum(w_ref[...], axis=0, keepdims=True)
    part = jax.lax.dot_general(
        x_ref[...], wsum,
        dimension_numbers=(((1,), (1,)), ((), ())),
        preferred_element_type=jnp.float32,
    )

    @pl.when(k == 0)
    def _init():
        o_ref[...] = (part + jnp.sum(b_ref[...]))[None]

    @pl.when(k > 0)
    def _acc():
        o_ref[...] += part[None]


def kernel(x, W, b):
    b3 = b.reshape(NCORES, 1, I // NCORES)
    partials = pl.pallas_call(
        _body,
        grid=(NCORES, KC),
        in_specs=[
            pl.BlockSpec((B, BLK), lambda c, k: (0, c * KC + k)),
            pl.BlockSpec((B, BLK), lambda c, k: (0, c * KC + k)),
            pl.BlockSpec((1, 1, I // NCORES), lambda c, k: (c, 0, 0)),
        ],
        out_specs=pl.BlockSpec((1, B, 1), lambda c, k: (c, 0, 0)),
        out_shape=jax.ShapeDtypeStruct((NCORES, B, 1), jnp.float32),
        compiler_params=pltpu.CompilerParams(
            dimension_semantics=("parallel", "arbitrary"),
        ),
    )(x, W, b3)
    return partials[0] + partials[1]
